# baseline (speedup 1.0000x reference)
import functools

import jax
import jax.numpy as jnp
from jax import lax
from jax.experimental import pallas as pl
from jax.experimental.pallas import tpu as pltpu
from jax.experimental.pallas import tpu_sc as plsc

_NC = 2
_NS = 16
_L = 16
_NW = _NC * _NS
_CH = 32
_SUB = 8


@functools.lru_cache(maxsize=None)
def _make_k1(B, D):
    bpw = B // _NW
    nchunk = bpw // _CH
    mesh = plsc.VectorSubcoreMesh(core_axis_name="c", subcore_axis_name="s")

    @functools.partial(
        pl.kernel,
        out_type=jax.ShapeDtypeStruct((B * D,), jnp.float32),
        mesh=mesh,
        scratch_types=[
            pltpu.VMEM((bpw,), jnp.int32),
            pltpu.VMEM((bpw,), jnp.int32),
            pltpu.VMEM((_CH, _SUB, D), jnp.float32),
            pltpu.VMEM((bpw * D,), jnp.float32),
            pltpu.SemaphoreType.DMA,
            pltpu.SemaphoreType.DMA,
        ],
        compiler_params=pltpu.CompilerParams(
            needs_layout_passes=False, use_tc_tiling_on_sc=True),
    )
    def k1(item_hbm, iid_hbm, out_hbm,
           iidx_v, itid_v, buf_v, rows_v, sem0, sem1):
        wid = lax.axis_index("s") * _NC + lax.axis_index("c")
        base = wid * bpw
        pltpu.sync_copy(iid_hbm.at[pl.ds(base, bpw)], iidx_v)

        def tids(s, carry):
            ivec = iidx_v[pl.ds(s * _L, _L)]
            itid_v[pl.ds(s * _L, _L)] = lax.shift_right_logical(ivec, 3)
            return carry

        lax.fori_loop(0, bpw // _L, tids, 0)

        def chunk_body(g, carry):
            descs = []
            for sub in range(_CH // _L):
                k0 = g * _CH + sub * _L
                itvec = itid_v[pl.ds(k0, _L)]
                for j in range(_L):
                    m = sub * _L + j
                    descs.append(pltpu.async_copy(
                        item_hbm.at[itvec[j]], buf_v.at[m],
                        sem0 if m % 2 == 0 else sem1))
            for d in descs:
                d.wait()

            for sub in range(_CH // _L):
                k0 = g * _CH + sub * _L
                ivec = jnp.bitwise_and(iidx_v[pl.ds(k0, _L)], 7)
                for j in range(_L):
                    m = sub * _L + j
                    ri = ivec[j]
                    r = k0 + j
                    for c in range(D // _L):
                        rows_v[pl.ds(r * D + c * _L, _L)] = (
                            buf_v[m, ri, pl.ds(c * _L, _L)])
            return carry

        lax.fori_loop(0, nchunk, chunk_body, 0)
        pltpu.sync_copy(rows_v, out_hbm.at[pl.ds(base * D, bpw * D)])

    return k1


@functools.lru_cache(maxsize=None)
def _make_k2(B, D):
    bpw = B // _NW
    mesh = plsc.VectorSubcoreMesh(core_axis_name="c", subcore_axis_name="s")

    @functools.partial(
        pl.kernel,
        out_type=jax.ShapeDtypeStruct((B,), jnp.float32),
        mesh=mesh,
        scratch_types=[
            pltpu.VMEM((bpw,), jnp.int32),
            pltpu.VMEM((bpw, D), jnp.float32),
            pltpu.VMEM((bpw * D,), jnp.float32),
            pltpu.VMEM((bpw,), jnp.float32),
            pltpu.SemaphoreType.DMA,
            pltpu.SemaphoreType.DMA,
        ],
        compiler_params=pltpu.CompilerParams(
            needs_layout_passes=False, use_tc_tiling_on_sc=False),
    )
    def k2(user_hbm, brows_hbm, uid_hbm, out_hbm,
           uidx_v, urows_v, brows_v, out_v, usem, bsem):
        wid = lax.axis_index("s") * _NC + lax.axis_index("c")
        base = wid * bpw
        pltpu.sync_copy(uid_hbm.at[pl.ds(base, bpw)], uidx_v)
        cu = pltpu.async_copy(user_hbm.at[uidx_v], urows_v, usem)
        cb = pltpu.async_copy(
            brows_hbm.at[pl.ds(base * D, bpw * D)], brows_v, bsem)
        cu.wait()
        cb.wait()

        lane = lax.iota(jnp.int32, _L)

        def group(g, carry):
            res = jnp.zeros((_L,), jnp.float32)
            for j in range(_L):
                r = g * _L + j
                acc = (urows_v[r, pl.ds(0, _L)]
                       * brows_v[pl.ds(r * D, _L)])
                for c in range(1, D // _L):
                    acc = acc + (urows_v[r, pl.ds(c * _L, _L)]
                                 * brows_v[pl.ds(r * D + c * _L, _L)])
                s = jnp.sum(acc)
                res = jnp.where(lane == j, s, res)
            out_v[pl.ds(g * _L, _L)] = res
            return carry

        lax.fori_loop(0, bpw // _L, group, 0)
        pltpu.sync_copy(out_v, out_hbm.at[pl.ds(base, bpw)])

    return k2


def kernel(user_table, item_table, user_ids, item_ids):
    B = user_ids.shape[0]
    N, D = user_table.shape
    M = item_table.shape[0]
    i3 = item_table.reshape(M // _SUB, _SUB, D)
    brows = _make_k1(B, D)(i3, item_ids.astype(jnp.int32))
    return _make_k2(B, D)(user_table, brows, user_ids.astype(jnp.int32))


# speedup vs baseline: 1.1231x; 1.1231x over previous
import functools

import jax
import jax.numpy as jnp
from jax import lax
from jax.experimental import pallas as pl
from jax.experimental.pallas import tpu as pltpu
from jax.experimental.pallas import tpu_sc as plsc

_NC = 2
_NS = 16
_L = 16
_NW = _NC * _NS
_CH = 32
_SUB = 8


@functools.lru_cache(maxsize=None)
def _make_kernel(B, D):
    assert B % (8 * _NW) == 0 and D % _L == 0
    bpw = B // _NW
    nchunk = bpw // _CH
    mesh = plsc.VectorSubcoreMesh(core_axis_name="c", subcore_axis_name="s")

    @functools.partial(
        pl.kernel,
        out_type=jax.ShapeDtypeStruct((B,), jnp.float32),
        mesh=mesh,
        scratch_types=[
            pltpu.VMEM((bpw,), jnp.int32),
            pltpu.VMEM((bpw,), jnp.int32),
            pltpu.VMEM((bpw,), jnp.int32),
            pltpu.VMEM((bpw,), jnp.int32),
            pltpu.VMEM((_CH * _SUB, D), jnp.float32),
            pltpu.VMEM((_CH * _SUB, D), jnp.float32),
            pltpu.VMEM((bpw,), jnp.float32),
            pltpu.SemaphoreType.DMA,
            pltpu.SemaphoreType.DMA,
        ],
        compiler_params=pltpu.CompilerParams(
            needs_layout_passes=False, use_tc_tiling_on_sc=True),
    )
    def scores_kernel(user_hbm, item_hbm, uid_hbm, iid_hbm, out_hbm,
                      uidx_v, iidx_v, utid_v, itid_v, ubuf_v, vbuf_v,
                      out_v, usem, vsem):
        wid = lax.axis_index("s") * _NC + lax.axis_index("c")
        base = wid * bpw
        pltpu.sync_copy(uid_hbm.at[pl.ds(base, bpw)], uidx_v)
        pltpu.sync_copy(iid_hbm.at[pl.ds(base, bpw)], iidx_v)

        def tids(s, carry):
            uvec = uidx_v[pl.ds(s * _L, _L)]
            ivec = iidx_v[pl.ds(s * _L, _L)]
            utid_v[pl.ds(s * _L, _L)] = jnp.bitwise_and(uvec, ~7)
            itid_v[pl.ds(s * _L, _L)] = jnp.bitwise_and(ivec, ~7)
            return carry

        lax.fori_loop(0, bpw // _L, tids, 0)

        lane = lax.iota(jnp.int32, _L)

        def chunk_body(g, carry):
            descs = []
            for sub in range(_CH // _L):
                k0 = g * _CH + sub * _L
                utvec = utid_v[pl.ds(k0, _L)]
                itvec = itid_v[pl.ds(k0, _L)]
                for j in range(_L):
                    m = sub * _L + j
                    ub = pl.multiple_of(utvec[j], _SUB)
                    ib = pl.multiple_of(itvec[j], _SUB)
                    descs.append(pltpu.async_copy(
                        user_hbm.at[pl.ds(ub, _SUB), :],
                        ubuf_v.at[pl.ds(m * _SUB, _SUB), :], usem))
                    descs.append(pltpu.async_copy(
                        item_hbm.at[pl.ds(ib, _SUB), :],
                        vbuf_v.at[pl.ds(m * _SUB, _SUB), :], vsem))
            for d in descs:
                d.wait()

            for sub in range(_CH // _L):
                k0 = g * _CH + sub * _L
                uvec = jnp.bitwise_and(uidx_v[pl.ds(k0, _L)], 7)
                ivec = jnp.bitwise_and(iidx_v[pl.ds(k0, _L)], 7)
                res = jnp.zeros((_L,), jnp.float32)
                for j in range(_L):
                    m = sub * _L + j
                    ru = m * _SUB + uvec[j]
                    ri = m * _SUB + ivec[j]
                    acc = (ubuf_v[ru, pl.ds(0, _L)]
                           * vbuf_v[ri, pl.ds(0, _L)])
                    for c in range(1, D // _L):
                        acc = acc + (ubuf_v[ru, pl.ds(c * _L, _L)]
                                     * vbuf_v[ri, pl.ds(c * _L, _L)])
                    s = jnp.sum(acc)
                    res = jnp.where(lane == j, s, res)
                out_v[pl.ds(k0, _L)] = res
            return carry

        lax.fori_loop(0, nchunk, chunk_body, 0)
        pltpu.sync_copy(out_v, out_hbm.at[pl.ds(base, bpw)])

    return scores_kernel


def kernel(user_table, item_table, user_ids, item_ids):
    B = user_ids.shape[0]
    D = user_table.shape[1]
    f = _make_kernel(B, D)
    return f(user_table, item_table,
             user_ids.astype(jnp.int32), item_ids.astype(jnp.int32))


# speedup vs baseline: 1.6523x; 1.4712x over previous
import functools

import jax
import jax.numpy as jnp
from jax import lax
from jax.experimental import pallas as pl
from jax.experimental.pallas import tpu as pltpu
from jax.experimental.pallas import tpu_sc as plsc

_NC = 2
_NS = 16
_L = 16
_NW = _NC * _NS
_CH = 16
_SUB = 8


@functools.lru_cache(maxsize=None)
def _make_kernel(B, D):
    assert B % (8 * _NW) == 0 and D % _L == 0
    bpw = B // _NW
    npair = bpw // (2 * _CH)
    mesh = plsc.VectorSubcoreMesh(core_axis_name="c", subcore_axis_name="s")

    @functools.partial(
        pl.kernel,
        out_type=jax.ShapeDtypeStruct((B,), jnp.float32),
        mesh=mesh,
        scratch_types=[
            pltpu.VMEM((bpw,), jnp.int32),
            pltpu.VMEM((bpw,), jnp.int32),
            pltpu.VMEM((bpw,), jnp.int32),
            pltpu.VMEM((bpw,), jnp.int32),
            pltpu.VMEM((_CH, _SUB, D), jnp.float32),
            pltpu.VMEM((_CH, _SUB, D), jnp.float32),
            pltpu.VMEM((_CH, _SUB, D), jnp.float32),
            pltpu.VMEM((_CH, _SUB, D), jnp.float32),
            pltpu.VMEM((bpw,), jnp.float32),
            pltpu.SemaphoreType.DMA,
            pltpu.SemaphoreType.DMA,
            pltpu.SemaphoreType.DMA,
            pltpu.SemaphoreType.DMA,
        ],
        compiler_params=pltpu.CompilerParams(
            needs_layout_passes=False, use_tc_tiling_on_sc=True),
    )
    def scores_kernel(user_hbm, item_hbm, uid_hbm, iid_hbm, out_hbm,
                      uidx_v, iidx_v, utid_v, itid_v,
                      ubufa_v, vbufa_v, ubufb_v, vbufb_v,
                      out_v, usema, vsema, usemb, vsemb):
        wid = lax.axis_index("s") * _NC + lax.axis_index("c")
        base = wid * bpw
        pltpu.sync_copy(uid_hbm.at[pl.ds(base, bpw)], uidx_v)
        pltpu.sync_copy(iid_hbm.at[pl.ds(base, bpw)], iidx_v)

        def tids(s, carry):
            uvec = uidx_v[pl.ds(s * _L, _L)]
            ivec = iidx_v[pl.ds(s * _L, _L)]
            utid_v[pl.ds(s * _L, _L)] = lax.shift_right_logical(uvec, 3)
            itid_v[pl.ds(s * _L, _L)] = lax.shift_right_logical(ivec, 3)
            return carry

        lax.fori_loop(0, bpw // _L, tids, 0)

        lane = lax.iota(jnp.int32, _L)

        def issue(g, ubuf, vbuf, usem, vsem):
            descs = []
            utvec = utid_v[pl.ds(g * _CH, _L)]
            itvec = itid_v[pl.ds(g * _CH, _L)]
            for j in range(_L):
                descs.append(pltpu.async_copy(
                    user_hbm.at[utvec[j]], ubuf.at[j], usem))
                descs.append(pltpu.async_copy(
                    item_hbm.at[itvec[j]], vbuf.at[j], vsem))
            return descs

        def compute(g, ubuf, vbuf):
            k0 = g * _CH
            uvec = jnp.bitwise_and(uidx_v[pl.ds(k0, _L)], 7)
            ivec = jnp.bitwise_and(iidx_v[pl.ds(k0, _L)], 7)
            res = jnp.zeros((_L,), jnp.float32)
            for j in range(_L):
                ru = uvec[j]
                ri = ivec[j]
                acc = (ubuf[j, ru, pl.ds(0, _L)]
                       * vbuf[j, ri, pl.ds(0, _L)])
                for c in range(1, D // _L):
                    acc = acc + (ubuf[j, ru, pl.ds(c * _L, _L)]
                                 * vbuf[j, ri, pl.ds(c * _L, _L)])
                s = jnp.sum(acc)
                res = jnp.where(lane == j, s, res)
            out_v[pl.ds(k0, _L)] = res

        def pair_body(gp, carry):
            ga = 2 * gp
            gb = 2 * gp + 1
            da = issue(ga, ubufa_v, vbufa_v, usema, vsema)
            db = issue(gb, ubufb_v, vbufb_v, usemb, vsemb)
            for d in da:
                d.wait()
            compute(ga, ubufa_v, vbufa_v)
            for d in db:
                d.wait()
            compute(gb, ubufb_v, vbufb_v)
            return carry

        lax.fori_loop(0, npair, pair_body, 0)
        pltpu.sync_copy(out_v, out_hbm.at[pl.ds(base, bpw)])

    return scores_kernel


def kernel(user_table, item_table, user_ids, item_ids):
    B = user_ids.shape[0]
    N, D = user_table.shape
    M = item_table.shape[0]
    u3 = user_table.reshape(N // _SUB, _SUB, D)
    i3 = item_table.reshape(M // _SUB, _SUB, D)
    f = _make_kernel(B, D)
    return f(u3, i3, user_ids.astype(jnp.int32), item_ids.astype(jnp.int32))


# speedup vs baseline: 1.6562x; 1.0023x over previous
import functools

import jax
import jax.numpy as jnp
from jax import lax
from jax.experimental import pallas as pl
from jax.experimental.pallas import tpu as pltpu
from jax.experimental.pallas import tpu_sc as plsc

_NC = 2
_NS = 16
_L = 16
_NW = _NC * _NS
_CH = 32
_SUB = 8


@functools.lru_cache(maxsize=None)
def _make_kernel(B, D):
    assert B % (8 * _NW) == 0 and D % _L == 0
    bpw = B // _NW
    nchunk = bpw // _CH
    mesh = plsc.VectorSubcoreMesh(core_axis_name="c", subcore_axis_name="s")

    @functools.partial(
        pl.kernel,
        out_type=jax.ShapeDtypeStruct((B,), jnp.float32),
        mesh=mesh,
        scratch_types=[
            pltpu.VMEM((bpw,), jnp.int32),
            pltpu.VMEM((bpw,), jnp.int32),
            pltpu.VMEM((bpw,), jnp.int32),
            pltpu.VMEM((bpw,), jnp.int32),
            pltpu.VMEM((_CH, _SUB, D), jnp.float32),
            pltpu.VMEM((_CH, _SUB, D), jnp.float32),
            pltpu.VMEM((bpw,), jnp.float32),
            pltpu.SemaphoreType.DMA,
            pltpu.SemaphoreType.DMA,
            pltpu.SemaphoreType.DMA,
            pltpu.SemaphoreType.DMA,
        ],
        compiler_params=pltpu.CompilerParams(
            needs_layout_passes=False, use_tc_tiling_on_sc=True),
    )
    def scores_kernel(user_hbm, item_hbm, uid_hbm, iid_hbm, out_hbm,
                      uidx_v, iidx_v, utid_v, itid_v, ubuf_v, vbuf_v,
                      out_v, sem0, sem1, sem2, sem3):
        sems = [sem0, sem1, sem2, sem3]
        wid = lax.axis_index("s") * _NC + lax.axis_index("c")
        base = wid * bpw
        pltpu.sync_copy(uid_hbm.at[pl.ds(base, bpw)], uidx_v)
        pltpu.sync_copy(iid_hbm.at[pl.ds(base, bpw)], iidx_v)

        def tids(s, carry):
            uvec = uidx_v[pl.ds(s * _L, _L)]
            ivec = iidx_v[pl.ds(s * _L, _L)]
            utid_v[pl.ds(s * _L, _L)] = lax.shift_right_logical(uvec, 3)
            itid_v[pl.ds(s * _L, _L)] = lax.shift_right_logical(ivec, 3)
            return carry

        lax.fori_loop(0, bpw // _L, tids, 0)

        lane = lax.iota(jnp.int32, _L)

        def chunk_body(g, carry):
            descs = []
            for sub in range(_CH // _L):
                k0 = g * _CH + sub * _L
                utvec = utid_v[pl.ds(k0, _L)]
                itvec = itid_v[pl.ds(k0, _L)]
                for j in range(_L):
                    m = sub * _L + j
                    descs.append(pltpu.async_copy(
                        user_hbm.at[utvec[j]], ubuf_v.at[m],
                        sems[(2 * m) % 4]))
                    descs.append(pltpu.async_copy(
                        item_hbm.at[itvec[j]], vbuf_v.at[m],
                        sems[(2 * m + 1) % 4]))
            for d in descs:
                d.wait()

            for sub in range(_CH // _L):
                k0 = g * _CH + sub * _L
                uvec = jnp.bitwise_and(uidx_v[pl.ds(k0, _L)], 7)
                ivec = jnp.bitwise_and(iidx_v[pl.ds(k0, _L)], 7)
                res = jnp.zeros((_L,), jnp.float32)
                for j in range(_L):
                    m = sub * _L + j
                    ru = uvec[j]
                    ri = ivec[j]
                    acc = (ubuf_v[m, ru, pl.ds(0, _L)]
                           * vbuf_v[m, ri, pl.ds(0, _L)])
                    for c in range(1, D // _L):
                        acc = acc + (ubuf_v[m, ru, pl.ds(c * _L, _L)]
                                     * vbuf_v[m, ri, pl.ds(c * _L, _L)])
                    s = jnp.sum(acc)
                    res = jnp.where(lane == j, s, res)
                out_v[pl.ds(k0, _L)] = res
            return carry

        lax.fori_loop(0, nchunk, chunk_body, 0)
        pltpu.sync_copy(out_v, out_hbm.at[pl.ds(base, bpw)])

    return scores_kernel


def kernel(user_table, item_table, user_ids, item_ids):
    B = user_ids.shape[0]
    N, D = user_table.shape
    M = item_table.shape[0]
    u3 = user_table.reshape(N // _SUB, _SUB, D)
    i3 = item_table.reshape(M // _SUB, _SUB, D)
    f = _make_kernel(B, D)
    return f(u3, i3, user_ids.astype(jnp.int32), item_ids.astype(jnp.int32))
